# baseline (speedup 1.0000x reference)
import functools

import jax
import jax.numpy as jnp
from jax import lax
from jax.experimental import pallas as pl
from jax.experimental.pallas import tpu as pltpu
from jax.experimental.pallas import tpu_sc as plsc

_B, _E, _S, _F = 16, 1024, 256, 8
_A, _C = 16, 8
_ESC = 256
_ETC = _E - _ESC
_EC = 32
_NCH = _ESC // _EC
_SH = 128


def _tc_body(scale_ref, alpha_ref, m_ref, eq_ref, x_ref, o_ref):
    m = m_ref[...]
    eq = eq_ref[...]
    scale = scale_ref[0]
    alpha = alpha_ref[0]

    w1 = jax.lax.dot_general(m, eq, (((0,), (0,)), ((), ())),
                             preferred_element_type=jnp.float32)
    w1s = w1 * (-scale)
    w2s = (jnp.sum(eq, axis=0)[None, :] - w1) * alpha

    t = x_ref[0]
    xs = t.sum(axis=1)
    obs = jnp.logical_not(jnp.isnan(xs))
    xc = jnp.where(obs, xs, 0.0)
    obs_f = obs.astype(jnp.float32)

    part = jax.lax.dot_general(xc, w1s, (((0,), (0,)), ((), ())),
                               preferred_element_type=jnp.float32)
    part += jax.lax.dot_general(obs_f, w2s, (((0,), (0,)), ((), ())),
                                preferred_element_type=jnp.float32)
    o_ref[0] = part


def _tc_part(scale1, alpha1, m_f, eq_classes, xt):
    return pl.pallas_call(
        _tc_body,
        grid=(_B,),
        in_specs=[
            pl.BlockSpec(memory_space=pltpu.SMEM),
            pl.BlockSpec(memory_space=pltpu.SMEM),
            pl.BlockSpec((_A, _ETC), lambda b: (0, 0)),
            pl.BlockSpec((_A, _C), lambda b: (0, 0)),
            pl.BlockSpec((1, _ETC, _F, _S), lambda b: (b, 0, 0, 0)),
        ],
        out_specs=pl.BlockSpec((1, _S, _C), lambda b: (b, 0, 0)),
        out_shape=jax.ShapeDtypeStruct((_B, _S, _C), jnp.float32),
        compiler_params=pltpu.CompilerParams(
            dimension_semantics=("parallel",),
        ),
    )(scale1, alpha1, m_f, eq_classes, xt)


def _sc_body(x5, m_hbm, eq_hbm, sa_hbm, out_hbm,
             buf, m_v, eq_v, sa_v, w1_v, w2_v, acc_v, stg_v, sem):
    cid = lax.axis_index("c")
    sid = lax.axis_index("s")
    w = sid * 2 + cid
    b = w // 2
    shalf = w % 2

    pltpu.sync_copy(m_hbm.at[:, pl.ds(_ETC, _ESC)], m_v)
    pltpu.sync_copy(eq_hbm, eq_v)
    pltpu.sync_copy(sa_hbm, sa_v)
    sav = sa_v[...]
    scale = sav[0]
    alpha = sav[1]

    eq_s = []
    for a in range(_A):
        row = eq_v[a]
        eq_s.append([row[c] for c in range(_C)])
    eqcol = []
    for c in range(_C):
        s = eq_s[0][c]
        for a in range(1, _A):
            s = s + eq_s[a][c]
        eqcol.append(s)

    def w_body(ev, carry):
        for c in range(_C):
            acc = m_v[0, pl.ds(ev * 16, 16)] * eq_s[0][c]
            for a in range(1, _A):
                acc = acc + m_v[a, pl.ds(ev * 16, 16)] * eq_s[a][c]
            w1_v[pl.ds(c * _ESC + ev * 16, 16)] = acc * (-scale)
            w2_v[pl.ds(c * _ESC + ev * 16, 16)] = (eqcol[c] - acc) * alpha
        return carry

    lax.fori_loop(0, _ESC // 16, w_body, 0)

    zz = jnp.zeros((16,), jnp.float32)
    for c in range(_C):
        for g in range(_SH // 16):
            acc_v[c, pl.ds(g * 16, 16)] = zz

    def chunk_body(i, carry):
        e0 = _ETC + i * _EC
        pltpu.sync_copy(x5.at[b, pl.ds(e0, _EC), shalf], buf)

        def e_body(j, carry2):
            el = jnp.full((16,), i * _EC + j, jnp.int32)
            w1b = [plsc.load_gather(w1_v, [el + (c * _ESC)])
                   for c in range(_C)]
            w2b = [plsc.load_gather(w2_v, [el + (c * _ESC)])
                   for c in range(_C)]
            for g in range(_SH // 16):
                xv = buf[j, 0, pl.ds(g * 16, 16)]
                for f in range(1, _F):
                    xv = xv + buf[j, f, pl.ds(g * 16, 16)]
                msk = xv == xv
                for c in range(_C):
                    t = xv * w1b[c] + w2b[c]
                    t = jnp.where(msk, t, 0.0)
                    plsc.addupdate(acc_v.at[c, pl.ds(g * 16, 16)], t)
            return carry2

        return lax.fori_loop(0, _EC, e_body, carry)

    lax.fori_loop(0, _NCH, chunk_body, 0)

    for g in range(_SH // 16):
        rows = lax.iota(jnp.int32, 16) + (g * 16)
        for c in range(_C):
            v = acc_v[c, pl.ds(g * 16, 16)]
            plsc.store_scatter(stg_v, [rows * _C + c], v)
    pltpu.sync_copy(stg_v, out_hbm.at[b, pl.ds(shalf * _SH * _C, _SH * _C)])


_sc_part = functools.partial(
    pl.kernel,
    out_type=jax.ShapeDtypeStruct((_B, _S * _C), jnp.float32),
    mesh=plsc.VectorSubcoreMesh(core_axis_name="c", subcore_axis_name="s"),
    scratch_types=[
        pltpu.VMEM((_EC, _F, _SH), jnp.float32),
        pltpu.VMEM((_A, _ESC), jnp.float32),
        pltpu.VMEM((_A, 16), jnp.float32),
        pltpu.VMEM((16,), jnp.float32),
        pltpu.VMEM((_C * _ESC,), jnp.float32),
        pltpu.VMEM((_C * _ESC,), jnp.float32),
        pltpu.VMEM((_C, _SH), jnp.float32),
        pltpu.VMEM((_SH * _C,), jnp.float32),
        pltpu.SemaphoreType.DMA,
    ],
    compiler_params=pltpu.CompilerParams(needs_layout_passes=False),
)(_sc_body)


@jax.jit
def kernel(input_seq, eq_classes, scale, alpha, edge_masks):
    xt = jnp.transpose(input_seq, (0, 1, 3, 2))
    x5 = jnp.transpose(
        input_seq.reshape(_B, _E, 2, _SH, _F), (0, 1, 2, 4, 3)
    )
    m_f = edge_masks.astype(jnp.float32)
    sa = jnp.concatenate(
        [scale.reshape(1), alpha.reshape(1), jnp.zeros((14,), jnp.float32)]
    )
    eq_pad = jnp.pad(eq_classes, ((0, 0), (0, 16 - _C)))
    out_tc = _tc_part(scale.reshape(1), alpha.reshape(1), m_f, eq_classes, xt)
    out_sc = _sc_part(x5, m_f, eq_pad, sa)
    return out_tc + out_sc.reshape(_B, _S, _C)


# speedup vs baseline: 1.0517x; 1.0517x over previous
import functools

import jax
import jax.numpy as jnp
from jax import lax
from jax.experimental import pallas as pl
from jax.experimental.pallas import tpu as pltpu
from jax.experimental.pallas import tpu_sc as plsc

_B, _E, _S, _F = 16, 1024, 256, 8
_A, _C = 16, 8
_ESC = 128
_ETC = _E - _ESC
_EC = 32
_NCH = _ESC // _EC
_SH = 128


def _tc_body(scale_ref, alpha_ref, m_ref, eq_ref, x_ref, o_ref):
    m = m_ref[...]
    eq = eq_ref[...]
    scale = scale_ref[0]
    alpha = alpha_ref[0]

    w1 = jax.lax.dot_general(m, eq, (((0,), (0,)), ((), ())),
                             preferred_element_type=jnp.float32)
    w1s = w1 * (-scale)
    w2s = (jnp.sum(eq, axis=0)[None, :] - w1) * alpha

    t = x_ref[0]
    xs = t.sum(axis=1)
    obs = jnp.logical_not(jnp.isnan(xs))
    xc = jnp.where(obs, xs, 0.0)
    obs_f = obs.astype(jnp.float32)

    part = jax.lax.dot_general(xc, w1s, (((0,), (0,)), ((), ())),
                               preferred_element_type=jnp.float32)
    part += jax.lax.dot_general(obs_f, w2s, (((0,), (0,)), ((), ())),
                                preferred_element_type=jnp.float32)
    o_ref[0] = part


def _tc_part(scale1, alpha1, m_f, eq_classes, xt):
    return pl.pallas_call(
        _tc_body,
        grid=(_B,),
        in_specs=[
            pl.BlockSpec(memory_space=pltpu.SMEM),
            pl.BlockSpec(memory_space=pltpu.SMEM),
            pl.BlockSpec((_A, _ETC), lambda b: (0, 0)),
            pl.BlockSpec((_A, _C), lambda b: (0, 0)),
            pl.BlockSpec((1, _ETC, _F, _S), lambda b: (b, 0, 0, 0)),
        ],
        out_specs=pl.BlockSpec((1, _S, _C), lambda b: (b, 0, 0)),
        out_shape=jax.ShapeDtypeStruct((_B, _S, _C), jnp.float32),
        compiler_params=pltpu.CompilerParams(
            dimension_semantics=("parallel",),
        ),
    )(scale1, alpha1, m_f, eq_classes, xt)


def _sc_body(x5, m_hbm, eq_hbm, sa_hbm, out_hbm,
             buf, m_v, eq_v, sa_v, w1_v, w2_v, acc_v, stg_v, sem):
    cid = lax.axis_index("c")
    sid = lax.axis_index("s")
    w = sid * 2 + cid
    b = w // 2
    shalf = w % 2

    pltpu.sync_copy(m_hbm.at[:, pl.ds(_ETC, _ESC)], m_v)
    pltpu.sync_copy(eq_hbm, eq_v)
    pltpu.sync_copy(sa_hbm, sa_v)
    sav = sa_v[...]
    scale = sav[0]
    alpha = sav[1]

    eq_s = []
    for a in range(_A):
        row = eq_v[a]
        eq_s.append([row[c] for c in range(_C)])
    eqcol = []
    for c in range(_C):
        s = eq_s[0][c]
        for a in range(1, _A):
            s = s + eq_s[a][c]
        eqcol.append(s)

    def w_body(ev, carry):
        for c in range(_C):
            acc = m_v[0, pl.ds(ev * 16, 16)] * eq_s[0][c]
            for a in range(1, _A):
                acc = acc + m_v[a, pl.ds(ev * 16, 16)] * eq_s[a][c]
            w1_v[pl.ds(c * _ESC + ev * 16, 16)] = acc * (-scale)
            w2_v[pl.ds(c * _ESC + ev * 16, 16)] = (eqcol[c] - acc) * alpha
        return carry

    lax.fori_loop(0, _ESC // 16, w_body, 0)

    zz = jnp.zeros((16,), jnp.float32)
    for c in range(_C):
        for g in range(_SH // 16):
            acc_v[c, pl.ds(g * 16, 16)] = zz

    def chunk_body(i, carry):
        e0 = _ETC + i * _EC
        pltpu.sync_copy(x5.at[b, pl.ds(e0, _EC), shalf], buf)

        def e_body(j, carry2):
            el = jnp.full((16,), i * _EC + j, jnp.int32)
            w1b = [plsc.load_gather(w1_v, [el + (c * _ESC)])
                   for c in range(_C)]
            w2b = [plsc.load_gather(w2_v, [el + (c * _ESC)])
                   for c in range(_C)]
            for g in range(_SH // 16):
                xv = buf[j, 0, pl.ds(g * 16, 16)]
                for f in range(1, _F):
                    xv = xv + buf[j, f, pl.ds(g * 16, 16)]
                msk = xv == xv
                for c in range(_C):
                    t = xv * w1b[c] + w2b[c]
                    t = jnp.where(msk, t, 0.0)
                    plsc.addupdate(acc_v.at[c, pl.ds(g * 16, 16)], t)
            return carry2

        return lax.fori_loop(0, _EC, e_body, carry)

    lax.fori_loop(0, _NCH, chunk_body, 0)

    for g in range(_SH // 16):
        rows = lax.iota(jnp.int32, 16) + (g * 16)
        for c in range(_C):
            v = acc_v[c, pl.ds(g * 16, 16)]
            plsc.store_scatter(stg_v, [rows * _C + c], v)
    pltpu.sync_copy(stg_v, out_hbm.at[b, pl.ds(shalf * _SH * _C, _SH * _C)])


_sc_part = functools.partial(
    pl.kernel,
    out_type=jax.ShapeDtypeStruct((_B, _S * _C), jnp.float32),
    mesh=plsc.VectorSubcoreMesh(core_axis_name="c", subcore_axis_name="s"),
    scratch_types=[
        pltpu.VMEM((_EC, _F, _SH), jnp.float32),
        pltpu.VMEM((_A, _ESC), jnp.float32),
        pltpu.VMEM((_A, 16), jnp.float32),
        pltpu.VMEM((16,), jnp.float32),
        pltpu.VMEM((_C * _ESC,), jnp.float32),
        pltpu.VMEM((_C * _ESC,), jnp.float32),
        pltpu.VMEM((_C, _SH), jnp.float32),
        pltpu.VMEM((_SH * _C,), jnp.float32),
        pltpu.SemaphoreType.DMA,
    ],
    compiler_params=pltpu.CompilerParams(needs_layout_passes=False),
)(_sc_body)


@jax.jit
def kernel(input_seq, eq_classes, scale, alpha, edge_masks):
    xt = jnp.transpose(input_seq, (0, 1, 3, 2))
    x5 = jnp.transpose(
        input_seq.reshape(_B, _E, 2, _SH, _F), (0, 1, 2, 4, 3)
    )
    m_f = edge_masks.astype(jnp.float32)
    sa = jnp.concatenate(
        [scale.reshape(1), alpha.reshape(1), jnp.zeros((14,), jnp.float32)]
    )
    eq_pad = jnp.pad(eq_classes, ((0, 0), (0, 16 - _C)))
    out_tc = _tc_part(scale.reshape(1), alpha.reshape(1), m_f, eq_classes, xt)
    out_sc = _sc_part(x5, m_f, eq_pad, sa)
    return out_tc + out_sc.reshape(_B, _S, _C)
